# baseline (speedup 1.0000x reference)
import functools

import jax
import jax.numpy as jnp
from jax import lax
from jax.experimental import pallas as pl
from jax.experimental.pallas import tpu as pltpu
from jax.experimental.pallas import tpu_sc as plsc

_N_CB = 4
_K = 1024
_BLK = 512
_QW = 128


def _dist_argmin(res, cb, blk_m, idx_ref):
    r2 = jnp.sum(res * res, axis=1, keepdims=True)
    c2 = jnp.sum(cb * cb, axis=1)
    dots = jax.lax.dot_general(
        res, cb, dimension_numbers=(((1,), (1,)), ((), ())),
        preferred_element_type=jnp.float32)
    dist = r2 - 2.0 * dots + c2[None, :]
    idx_ref[0, 0, :] = jnp.argmin(dist, axis=1).astype(jnp.int32)


def _round0_body(res_ref, cb_ref, idx_ref):
    _dist_argmin(res_ref[...], cb_ref[0], res_ref.shape[0], idx_ref)


def _round_body(res_ref, q_ref, cb_ref, res_out_ref, idx_ref):
    res = res_ref[...] - q_ref[:, :res_ref.shape[1]]
    res_out_ref[...] = res
    _dist_argmin(res, cb_ref[0], res.shape[0], idx_ref)


def _make_round0(n, d):
    nb = n // _BLK
    return pl.pallas_call(
        _round0_body,
        grid=(nb,),
        in_specs=[
            pl.BlockSpec((_BLK, d), lambda i: (i, 0)),
            pl.BlockSpec((1, _K, d), lambda i: (0, 0, 0)),
        ],
        out_specs=pl.BlockSpec((1, 1, _BLK), lambda i: (i, 0, 0)),
        out_shape=jax.ShapeDtypeStruct((nb, 1, _BLK), jnp.int32),
    )


def _make_round(n, d):
    nb = n // _BLK
    return pl.pallas_call(
        _round_body,
        grid=(nb,),
        in_specs=[
            pl.BlockSpec((_BLK, d), lambda i: (i, 0)),
            pl.BlockSpec((_BLK, _QW), lambda i: (i, 0)),
            pl.BlockSpec((1, _K, d), lambda i: (0, 0, 0)),
        ],
        out_specs=[
            pl.BlockSpec((_BLK, d), lambda i: (i, 0)),
            pl.BlockSpec((1, 1, _BLK), lambda i: (i, 0, 0)),
        ],
        out_shape=[
            jax.ShapeDtypeStruct((n, d), jnp.float32),
            jax.ShapeDtypeStruct((nb, 1, _BLK), jnp.int32),
        ],
    )


def _final_body(x_ref, res_ref, q_ref, o_ref):
    o_ref[...] = (x_ref[...] - res_ref[...]) + q_ref[:, :x_ref.shape[1]]


def _make_final(n, d):
    spec = pl.BlockSpec((_BLK, d), lambda i: (i, 0))
    return pl.pallas_call(
        _final_body,
        grid=(n // _BLK,),
        in_specs=[spec, spec, pl.BlockSpec((_BLK, _QW), lambda i: (i, 0))],
        out_specs=spec,
        out_shape=jax.ShapeDtypeStruct((n, d), jnp.float32),
    )


_CHUNK = 72
_NS = 4


def _make_gather(n):
    info = plsc.get_sparse_core_info()
    nw = info.num_cores * info.num_subcores
    b_per_w = n // nw
    n_ch = b_per_w // _CHUNK
    mesh = plsc.VectorSubcoreMesh(core_axis_name="c", subcore_axis_name="s")

    @functools.partial(
        pl.kernel, mesh=mesh,
        out_type=jax.ShapeDtypeStruct((n, _QW), jnp.float32),
        scratch_types=[
            pltpu.VMEM((n_ch, _CHUNK), jnp.int32),
            pltpu.VMEM((b_per_w, _QW), jnp.float32),
            pltpu.SemaphoreType.DMA,
        ],
    )
    def gather(table_hbm, idx_hbm, out_hbm, idx_v, rows_v, sem):
        wid = lax.axis_index("s") * info.num_cores + lax.axis_index("c")
        base = wid * b_per_w
        pltpu.sync_copy(idx_hbm.at[wid], idx_v)
        copies = [
            pltpu.async_copy(
                table_hbm.at[idx_v.at[j]],
                rows_v.at[pl.ds(j * _CHUNK, _CHUNK)], sem)
            for j in range(n_ch)
        ]
        for c in copies:
            c.wait()
        pltpu.sync_copy(rows_v, out_hbm.at[pl.ds(base, b_per_w)])

    return gather


def kernel(inputs, codebooks):
    b, t, d = inputs.shape
    n = b * t
    ns = n // _NS
    x = inputs.reshape(n, d)
    cbp = jnp.pad(codebooks, ((0, 0), (0, 0), (0, _QW - d)))
    round0 = _make_round0(ns, d)
    round_i = _make_round(ns, d)
    gather = _make_gather(ns)
    final = _make_final(ns, d)

    def gath(i, idx):
        return gather(cbp[i], idx.reshape(32, -1, _CHUNK))

    xs = [x[s * ns:(s + 1) * ns] for s in range(_NS)]
    idx = [round0(xh, codebooks[0:1]) for xh in xs]
    q = [gath(0, ih) for ih in idx]
    res = list(xs)
    for r in range(1, _N_CB):
        for s in range(_NS):
            res[s], idx[s] = round_i(res[s], q[s], codebooks[r:r + 1])
            q[s] = gath(r, idx[s])
    out = jnp.concatenate(
        [final(xs[s], res[s], q[s]) for s in range(_NS)], axis=0)
    return out.reshape(b, t, d)


# speedup vs baseline: 1.0380x; 1.0380x over previous
import functools

import jax
import jax.numpy as jnp
from jax import lax
from jax.experimental import pallas as pl
from jax.experimental.pallas import tpu as pltpu
from jax.experimental.pallas import tpu_sc as plsc

_N_CB = 4
_K = 1024
_BLK = 512
_QW = 128


def _dist_argmin(res, cb, blk_m, idx_ref):
    r2 = jnp.sum(res * res, axis=1, keepdims=True)
    c2 = jnp.sum(cb * cb, axis=1)
    dots = jax.lax.dot_general(
        res, cb, dimension_numbers=(((1,), (1,)), ((), ())),
        preferred_element_type=jnp.float32)
    dist = r2 - 2.0 * dots + c2[None, :]
    idx_ref[0, 0, :] = jnp.argmin(dist, axis=1).astype(jnp.int32)


def _round0_body(res_ref, cb_ref, idx_ref):
    _dist_argmin(res_ref[...], cb_ref[0], res_ref.shape[0], idx_ref)


def _round_body(res_ref, q_ref, cb_ref, res_out_ref, idx_ref):
    res = res_ref[...] - q_ref[:, :res_ref.shape[1]]
    res_out_ref[...] = res
    _dist_argmin(res, cb_ref[0], res.shape[0], idx_ref)


def _make_round0(n, d):
    nb = n // _BLK
    return pl.pallas_call(
        _round0_body,
        grid=(nb,),
        in_specs=[
            pl.BlockSpec((_BLK, d), lambda i: (i, 0)),
            pl.BlockSpec((1, _K, d), lambda i: (0, 0, 0)),
        ],
        out_specs=pl.BlockSpec((1, 1, _BLK), lambda i: (i, 0, 0)),
        out_shape=jax.ShapeDtypeStruct((nb, 1, _BLK), jnp.int32),
    )


def _make_round(n, d):
    nb = n // _BLK
    return pl.pallas_call(
        _round_body,
        grid=(nb,),
        in_specs=[
            pl.BlockSpec((_BLK, d), lambda i: (i, 0)),
            pl.BlockSpec((_BLK, _QW), lambda i: (i, 0)),
            pl.BlockSpec((1, _K, d), lambda i: (0, 0, 0)),
        ],
        out_specs=[
            pl.BlockSpec((_BLK, d), lambda i: (i, 0)),
            pl.BlockSpec((1, 1, _BLK), lambda i: (i, 0, 0)),
        ],
        out_shape=[
            jax.ShapeDtypeStruct((n, d), jnp.float32),
            jax.ShapeDtypeStruct((nb, 1, _BLK), jnp.int32),
        ],
    )


def _final_body(x_ref, res_ref, q_ref, o_ref):
    o_ref[...] = (x_ref[...] - res_ref[...]) + q_ref[:, :x_ref.shape[1]]


def _make_final(n, d):
    spec = pl.BlockSpec((_BLK, d), lambda i: (i, 0))
    return pl.pallas_call(
        _final_body,
        grid=(n // _BLK,),
        in_specs=[spec, spec, pl.BlockSpec((_BLK, _QW), lambda i: (i, 0))],
        out_specs=spec,
        out_shape=jax.ShapeDtypeStruct((n, d), jnp.float32),
    )


_CHUNK = 96


def _make_gather(n):
    info = plsc.get_sparse_core_info()
    nw = info.num_cores * info.num_subcores
    b_per_w = n // nw
    n_ch = b_per_w // _CHUNK
    mesh = plsc.VectorSubcoreMesh(core_axis_name="c", subcore_axis_name="s")

    @functools.partial(
        pl.kernel, mesh=mesh,
        out_type=jax.ShapeDtypeStruct((n, _QW), jnp.float32),
        scratch_types=[
            pltpu.VMEM((n_ch, _CHUNK), jnp.int32),
            pltpu.VMEM((b_per_w, _QW), jnp.float32),
            pltpu.SemaphoreType.DMA,
        ],
    )
    def gather(table_hbm, idx_hbm, out_hbm, idx_v, rows_v, sem):
        wid = lax.axis_index("s") * info.num_cores + lax.axis_index("c")
        base = wid * b_per_w
        pltpu.sync_copy(idx_hbm.at[wid], idx_v)
        copies = [
            pltpu.async_copy(
                table_hbm.at[idx_v.at[j]],
                rows_v.at[pl.ds(j * _CHUNK, _CHUNK)], sem)
            for j in range(n_ch)
        ]
        for c in copies:
            c.wait()
        pltpu.sync_copy(rows_v, out_hbm.at[pl.ds(base, b_per_w)])

    return gather


def kernel(inputs, codebooks):
    b, t, d = inputs.shape
    n = b * t
    n2 = n // 3
    x = inputs.reshape(n, d)
    cbp = jnp.pad(codebooks, ((0, 0), (0, 0), (0, _QW - d)))
    round0 = _make_round0(n2, d)
    round_i = _make_round(n2, d)
    gather = _make_gather(n2)
    final = _make_final(n2, d)

    def gath(i, idx):
        return gather(cbp[i], idx.reshape(32, -1, _CHUNK))

    nstream = n // n2
    xs = [x[s * n2:(s + 1) * n2] for s in range(nstream)]
    idx = [round0(xh, codebooks[0:1]) for xh in xs]
    q = [gath(0, ih) for ih in idx]
    res = list(xs)
    for r in range(1, _N_CB):
        for s in range(nstream):
            res[s], idx[s] = round_i(res[s], q[s], codebooks[r:r + 1])
        for s in range(nstream):
            q[s] = gath(r, idx[s])
    out = jnp.concatenate(
        [final(xs[s], res[s], q[s]) for s in range(nstream)], axis=0)
    return out.reshape(b, t, d)


# speedup vs baseline: 1.0847x; 1.0450x over previous
import functools

import jax
import jax.numpy as jnp
from jax import lax
from jax.experimental import pallas as pl
from jax.experimental.pallas import tpu as pltpu
from jax.experimental.pallas import tpu_sc as plsc

_N_CB = 4
_K = 1024
_BLK = 512
_QW = 128


def _dist_argmin(res, cb, blk_m, idx_ref):
    r2 = jnp.sum(res * res, axis=1, keepdims=True)
    c2 = jnp.sum(cb * cb, axis=1)
    dots = jax.lax.dot_general(
        res, cb, dimension_numbers=(((1,), (1,)), ((), ())),
        preferred_element_type=jnp.float32)
    dist = r2 - 2.0 * dots + c2[None, :]
    idx_ref[0, 0, :] = jnp.argmin(dist, axis=1).astype(jnp.int32)


def _round0_body(res_ref, cb_ref, idx_ref):
    _dist_argmin(res_ref[...], cb_ref[0], res_ref.shape[0], idx_ref)


def _round_body(res_ref, q_ref, cb_ref, res_out_ref, idx_ref):
    res = res_ref[...] - q_ref[:, :res_ref.shape[1]]
    res_out_ref[...] = res
    _dist_argmin(res, cb_ref[0], res.shape[0], idx_ref)


def _make_round0(n, d):
    nb = n // _BLK
    return pl.pallas_call(
        _round0_body,
        grid=(nb,),
        in_specs=[
            pl.BlockSpec((_BLK, d), lambda i: (i, 0)),
            pl.BlockSpec((1, _K, d), lambda i: (0, 0, 0)),
        ],
        out_specs=pl.BlockSpec((1, 1, _BLK), lambda i: (i, 0, 0)),
        out_shape=jax.ShapeDtypeStruct((nb, 1, _BLK), jnp.int32),
    )


def _make_round(n, d):
    nb = n // _BLK
    return pl.pallas_call(
        _round_body,
        grid=(nb,),
        in_specs=[
            pl.BlockSpec((_BLK, d), lambda i: (i, 0)),
            pl.BlockSpec((_BLK, _QW), lambda i: (i, 0)),
            pl.BlockSpec((1, _K, d), lambda i: (0, 0, 0)),
        ],
        out_specs=[
            pl.BlockSpec((_BLK, d), lambda i: (i, 0)),
            pl.BlockSpec((1, 1, _BLK), lambda i: (i, 0, 0)),
        ],
        out_shape=[
            jax.ShapeDtypeStruct((n, d), jnp.float32),
            jax.ShapeDtypeStruct((nb, 1, _BLK), jnp.int32),
        ],
    )


def _final_body(x_ref, res_ref, q_ref, o_ref):
    o_ref[...] = (x_ref[...] - res_ref[...]) + q_ref[:, :x_ref.shape[1]]


def _make_final(n, d):
    spec = pl.BlockSpec((_BLK, d), lambda i: (i, 0))
    return pl.pallas_call(
        _final_body,
        grid=(n // _BLK,),
        in_specs=[spec, spec, pl.BlockSpec((_BLK, _QW), lambda i: (i, 0))],
        out_specs=spec,
        out_shape=jax.ShapeDtypeStruct((n, d), jnp.float32),
    )


_CHUNK = 96


def _make_gather(n):
    info = plsc.get_sparse_core_info()
    nw = info.num_cores * info.num_subcores
    b_per_w = n // nw
    n_ch = b_per_w // _CHUNK
    mesh = plsc.VectorSubcoreMesh(core_axis_name="c", subcore_axis_name="s")

    @functools.partial(
        pl.kernel, mesh=mesh,
        out_type=jax.ShapeDtypeStruct((n, _QW), jnp.float32),
        scratch_types=[
            pltpu.VMEM((n_ch, _CHUNK), jnp.int32),
            pltpu.VMEM((b_per_w, _QW), jnp.float32),
            pltpu.SemaphoreType.DMA,
        ],
    )
    def gather(table_hbm, idx_hbm, out_hbm, idx_v, rows_v, sem):
        wid = lax.axis_index("s") * info.num_cores + lax.axis_index("c")
        base = wid * b_per_w
        pltpu.sync_copy(idx_hbm.at[wid], idx_v)
        copies = [
            pltpu.async_copy(
                table_hbm.at[idx_v.at[j]],
                rows_v.at[pl.ds(j * _CHUNK, _CHUNK)], sem)
            for j in range(n_ch)
        ]
        for c in copies:
            c.wait()
        pltpu.sync_copy(rows_v, out_hbm.at[pl.ds(base, b_per_w)])

    return gather


def kernel(inputs, codebooks):
    b, t, d = inputs.shape
    n = b * t
    n2 = n // 2
    x = inputs.reshape(n, d)
    cbp = jnp.pad(codebooks, ((0, 0), (0, 0), (0, _QW - d)))
    round0 = _make_round0(n2, d)
    round_i = _make_round(n2, d)
    gather = _make_gather(n2)
    final = _make_final(n2, d)

    def gath(i, idx):
        return gather(cbp[i], idx.reshape(32, -1, _CHUNK))

    nstream = n // n2
    xs = [x[s * n2:(s + 1) * n2] for s in range(nstream)]
    idx = [round0(xh, codebooks[0:1]) for xh in xs]
    q = [gath(0, ih) for ih in idx]
    res = list(xs)
    for r in range(1, _N_CB):
        for s in range(nstream):
            res[s], idx[s] = round_i(res[s], q[s], codebooks[r:r + 1])
        for s in range(nstream):
            q[s] = gath(r, idx[s])
    out = jnp.concatenate(
        [final(xs[s], res[s], q[s]) for s in range(nstream)], axis=0)
    return out.reshape(b, t, d)
